# Initial kernel scaffold; baseline (speedup 1.0000x reference)
import jax
import jax.numpy as jnp
from jax import lax
from jax.experimental import pallas as pl
from jax.experimental.pallas import tpu as pltpu
from jax.experimental.pallas import tpu_sc as plsc

N = 10000
E = 320000
ALPHA = 0.1
SWEEPS = 10
NT = 16
EPT = E // NT
VL = 16
NSL = 640
NP = NT * NSL


def _sweep_body(src_hbm, dst_hbm, w_hbm, h_out,
                src_v, dst_v, w_v, h_v, acc_v, red_v, hs_v, sh_acc, sh_h):
    wid = lax.axis_index("s")
    ebase = wid * EPT
    nbase = wid * NSL

    pltpu.sync_copy(src_hbm.at[pl.ds(ebase, EPT)], src_v)
    pltpu.sync_copy(dst_hbm.at[pl.ds(ebase, EPT)], dst_v)
    pltpu.sync_copy(w_hbm.at[pl.ds(ebase, EPT)], w_v)

    ones = jnp.ones((VL,), jnp.float32)
    zeros = jnp.zeros((VL,), jnp.float32)

    @pl.loop(0, NP // VL)
    def _init(i):
        off = i * VL
        h_v[pl.ds(off, VL)] = ones
        acc_v[pl.ds(off, VL)] = zeros

    @pl.loop(0, SWEEPS)
    def _sweep(_):
        @pl.loop(0, EPT // VL)
        def _edges(e):
            off = e * VL
            s = src_v[pl.ds(off, VL)]
            d = dst_v[pl.ds(off, VL)]
            ww = w_v[pl.ds(off, VL)]
            hv = plsc.load_gather(h_v, [s])
            plsc.addupdate_scatter(acc_v, [d], hv * ww)

        pltpu.sync_copy(acc_v, sh_acc.at[wid])
        plsc.subcore_barrier()

        for t in range(NT):
            pltpu.sync_copy(sh_acc.at[t, pl.ds(nbase, NSL)], red_v.at[t])

        @pl.loop(0, NSL // VL)
        def _upd(j):
            off = j * VL
            a = red_v[0, pl.ds(off, VL)]
            for t in range(1, NT):
                a = a + red_v[t, pl.ds(off, VL)]
            hs_v[pl.ds(off, VL)] = a * (1.0 - ALPHA) + ALPHA

        pltpu.sync_copy(hs_v, sh_h.at[pl.ds(nbase, NSL)])

        @pl.loop(0, NP // VL)
        def _clr(i):
            acc_v[pl.ds(i * VL, VL)] = zeros

        plsc.subcore_barrier()
        pltpu.sync_copy(sh_h, h_v)

    pltpu.sync_copy(hs_v, h_out.at[pl.ds(nbase, NSL)])


def _make_sweep():
    mesh = plsc.VectorSubcoreMesh(
        core_axis_name="c", subcore_axis_name="s", num_cores=1)
    return pl.kernel(
        _sweep_body,
        out_type=jax.ShapeDtypeStruct((NP,), jnp.float32),
        mesh=mesh,
        scratch_types=[
            pltpu.VMEM((EPT,), jnp.int32),
            pltpu.VMEM((EPT,), jnp.int32),
            pltpu.VMEM((EPT,), jnp.float32),
            pltpu.VMEM((NP,), jnp.float32),
            pltpu.VMEM((NP,), jnp.float32),
            pltpu.VMEM((NT, NSL), jnp.float32),
            pltpu.VMEM((NSL,), jnp.float32),
            pltpu.VMEM_SHARED((NT, NP), jnp.float32),
            pltpu.VMEM_SHARED((NP,), jnp.float32),
        ],
    )


def _divide_tc(f_ref, h_ref, o_ref):
    o_ref[...] = f_ref[...] / h_ref[...]


def kernel(features, edge_index, edge_weight):
    src = edge_index[0]
    dst = edge_index[1]

    h_pad = _make_sweep()(src, dst, edge_weight)
    h = h_pad[:N].reshape(N, 1)

    return pl.pallas_call(
        _divide_tc,
        out_shape=jax.ShapeDtypeStruct((N, features.shape[1]), jnp.float32),
        grid=(8,),
        in_specs=[
            pl.BlockSpec((N // 8, 128), lambda i: (i, 0)),
            pl.BlockSpec((N // 8, 1), lambda i: (i, 0)),
        ],
        out_specs=pl.BlockSpec((N // 8, 128), lambda i: (i, 0)),
    )(features, h)


# speedup vs baseline: 64.9693x; 64.9693x over previous
import jax
import jax.numpy as jnp
from jax import lax
from jax.experimental import pallas as pl
from jax.experimental.pallas import tpu as pltpu
from jax.experimental.pallas import tpu_sc as plsc

N = 10000
E = 320000
ALPHA = 0.1
SWEEPS = 10
NT = 16
EPT = E // NT
VL = 16
NSL = 640
NP = NT * NSL


def _sweep_body(src_hbm, dst_hbm, w_hbm, h_out,
                src_v, dst_v, w_v, h_v, acc_v, red_v, hs_v, sh_acc, sh_h):
    wid = lax.axis_index("s")
    ebase = wid * EPT
    nbase = wid * NSL

    pltpu.sync_copy(src_hbm.at[pl.ds(ebase, EPT)], src_v)
    pltpu.sync_copy(dst_hbm.at[pl.ds(ebase, EPT)], dst_v)
    pltpu.sync_copy(w_hbm.at[pl.ds(ebase, EPT)], w_v)

    ones = jnp.ones((VL,), jnp.float32)
    zeros = jnp.zeros((VL,), jnp.float32)

    @pl.loop(0, NP // VL)
    def _init(i):
        off = i * VL
        h_v[pl.ds(off, VL)] = ones
        acc_v[pl.ds(off, VL)] = zeros

    @pl.loop(0, SWEEPS)
    def _sweep(_):
        @pl.loop(0, EPT // VL)
        def _edges(e):
            off = e * VL
            s = src_v[pl.ds(off, VL)]
            d = dst_v[pl.ds(off, VL)]
            ww = w_v[pl.ds(off, VL)]
            hv = plsc.load_gather(h_v, [s])
            plsc.addupdate_scatter(acc_v, [d], hv * ww)

        pltpu.sync_copy(acc_v, sh_acc.at[wid])
        plsc.subcore_barrier()

        for t in range(NT):
            pltpu.sync_copy(sh_acc.at[t, pl.ds(nbase, NSL)], red_v.at[t])

        @pl.loop(0, NSL // VL)
        def _upd(j):
            off = j * VL
            a = red_v[0, pl.ds(off, VL)]
            for t in range(1, NT):
                a = a + red_v[t, pl.ds(off, VL)]
            hs_v[pl.ds(off, VL)] = a * (1.0 - ALPHA) + ALPHA

        pltpu.sync_copy(hs_v, sh_h.at[pl.ds(nbase, NSL)])

        @pl.loop(0, NP // VL)
        def _clr(i):
            acc_v[pl.ds(i * VL, VL)] = zeros

        plsc.subcore_barrier()
        pltpu.sync_copy(sh_h, h_v)

    pltpu.sync_copy(hs_v, h_out.at[pl.ds(nbase, NSL)])


def _make_sweep():
    mesh = plsc.VectorSubcoreMesh(
        core_axis_name="c", subcore_axis_name="s", num_cores=1)
    return pl.kernel(
        _sweep_body,
        out_type=jax.ShapeDtypeStruct((NP,), jnp.float32),
        mesh=mesh,
        scratch_types=[
            pltpu.VMEM((EPT,), jnp.int32),
            pltpu.VMEM((EPT,), jnp.int32),
            pltpu.VMEM((EPT,), jnp.float32),
            pltpu.VMEM((NP,), jnp.float32),
            pltpu.VMEM((NP,), jnp.float32),
            pltpu.VMEM((NT, NSL), jnp.float32),
            pltpu.VMEM((NSL,), jnp.float32),
            pltpu.VMEM_SHARED((NT, NP), jnp.float32),
            pltpu.VMEM_SHARED((NP,), jnp.float32),
        ],
        compiler_params=pltpu.CompilerParams(needs_layout_passes=False),
    )


def _divide_tc(f_ref, h_ref, o_ref):
    o_ref[...] = f_ref[...] / h_ref[...]


def kernel(features, edge_index, edge_weight):
    src = edge_index[0]
    dst = edge_index[1]

    h_pad = _make_sweep()(src, dst, edge_weight)
    h = h_pad[:N].reshape(N, 1)

    return pl.pallas_call(
        _divide_tc,
        out_shape=jax.ShapeDtypeStruct((N, features.shape[1]), jnp.float32),
        grid=(10,),
        in_specs=[
            pl.BlockSpec((N // 10, 128), lambda i: (i, 0)),
            pl.BlockSpec((N // 10, 1), lambda i: (i, 0)),
        ],
        out_specs=pl.BlockSpec((N // 10, 128), lambda i: (i, 0)),
    )(features, h)


# speedup vs baseline: 116.8654x; 1.7988x over previous
import jax
import jax.numpy as jnp
from jax import lax
from jax.experimental import pallas as pl
from jax.experimental.pallas import tpu as pltpu
from jax.experimental.pallas import tpu_sc as plsc

N = 10000
E = 320000
ALPHA = 0.1
SWEEPS = 10
NT = 16
EPT = E // NT
VL = 16
NSL = 640
NP = NT * NSL


def _sweep_body(src_hbm, dst_hbm, w_hbm, h_out,
                src_v, dst_v, w_v, h_v, acc_v, red_v, hs_v, sh_acc, sh_h):
    wid = lax.axis_index("s")
    ebase = wid * EPT
    nbase = wid * NSL

    pltpu.sync_copy(src_hbm.at[pl.ds(ebase, EPT)], src_v)
    pltpu.sync_copy(dst_hbm.at[pl.ds(ebase, EPT)], dst_v)
    pltpu.sync_copy(w_hbm.at[pl.ds(ebase, EPT)], w_v)

    ones = jnp.ones((VL,), jnp.float32)
    zeros = jnp.zeros((VL,), jnp.float32)

    @pl.loop(0, NP // VL)
    def _init(i):
        off = i * VL
        h_v[pl.ds(off, VL)] = ones
        acc_v[pl.ds(off, VL)] = zeros

    @pl.loop(0, SWEEPS)
    def _sweep(_):
        @plsc.parallel_loop(0, EPT // VL, unroll=8)
        def _edges(e):
            off = e * VL
            s = src_v[pl.ds(off, VL)]
            d = dst_v[pl.ds(off, VL)]
            ww = w_v[pl.ds(off, VL)]
            hv = plsc.load_gather(h_v, [s])
            plsc.addupdate_scatter(acc_v, [d], hv * ww)

        pltpu.sync_copy(acc_v, sh_acc.at[wid])
        plsc.subcore_barrier()

        for t in range(NT):
            pltpu.sync_copy(sh_acc.at[t, pl.ds(nbase, NSL)], red_v.at[t])

        @pl.loop(0, NSL // VL)
        def _upd(j):
            off = j * VL
            a = red_v[0, pl.ds(off, VL)]
            for t in range(1, NT):
                a = a + red_v[t, pl.ds(off, VL)]
            hs_v[pl.ds(off, VL)] = a * (1.0 - ALPHA) + ALPHA

        pltpu.sync_copy(hs_v, sh_h.at[pl.ds(nbase, NSL)])

        @plsc.parallel_loop(0, NP // VL, unroll=8)
        def _clr(i):
            acc_v[pl.ds(i * VL, VL)] = zeros

        plsc.subcore_barrier()
        pltpu.sync_copy(sh_h, h_v)

    pltpu.sync_copy(hs_v, h_out.at[pl.ds(nbase, NSL)])


def _make_sweep():
    mesh = plsc.VectorSubcoreMesh(
        core_axis_name="c", subcore_axis_name="s", num_cores=1)
    return pl.kernel(
        _sweep_body,
        out_type=jax.ShapeDtypeStruct((NP,), jnp.float32),
        mesh=mesh,
        scratch_types=[
            pltpu.VMEM((EPT,), jnp.int32),
            pltpu.VMEM((EPT,), jnp.int32),
            pltpu.VMEM((EPT,), jnp.float32),
            pltpu.VMEM((NP,), jnp.float32),
            pltpu.VMEM((NP,), jnp.float32),
            pltpu.VMEM((NT, NSL), jnp.float32),
            pltpu.VMEM((NSL,), jnp.float32),
            pltpu.VMEM_SHARED((NT, NP), jnp.float32),
            pltpu.VMEM_SHARED((NP,), jnp.float32),
        ],
        compiler_params=pltpu.CompilerParams(needs_layout_passes=False),
    )


def _divide_tc(f_ref, h_ref, o_ref):
    o_ref[...] = f_ref[...] / h_ref[...]


def kernel(features, edge_index, edge_weight):
    src = edge_index[0]
    dst = edge_index[1]

    h_pad = _make_sweep()(src, dst, edge_weight)
    h = h_pad[:N].reshape(N, 1)

    return pl.pallas_call(
        _divide_tc,
        out_shape=jax.ShapeDtypeStruct((N, features.shape[1]), jnp.float32),
        grid=(10,),
        in_specs=[
            pl.BlockSpec((N // 10, 128), lambda i: (i, 0)),
            pl.BlockSpec((N // 10, 1), lambda i: (i, 0)),
        ],
        out_specs=pl.BlockSpec((N // 10, 128), lambda i: (i, 0)),
    )(features, h)


# speedup vs baseline: 125.4757x; 1.0737x over previous
import jax
import jax.numpy as jnp
from jax import lax
from jax.experimental import pallas as pl
from jax.experimental.pallas import tpu as pltpu
from jax.experimental.pallas import tpu_sc as plsc

N = 10000
E = 320000
ALPHA = 0.1
SWEEPS = 10
NT = 16
EPT = E // NT
VL = 16
NSL = 640
NP = NT * NSL


def _sweep_body(ei_hbm, w_hbm, h_out,
                src_v, dst_v, w_v, h_v, acc_v, red_v, hs_v, sh_acc, sh_h):
    wid = lax.axis_index("s")
    ebase = wid * EPT
    nbase = wid * NSL

    pltpu.sync_copy(ei_hbm.at[pl.ds(ebase, EPT)], src_v)
    pltpu.sync_copy(ei_hbm.at[pl.ds(E + ebase, EPT)], dst_v)
    pltpu.sync_copy(w_hbm.at[pl.ds(ebase, EPT)], w_v)

    ones = jnp.ones((VL,), jnp.float32)
    zeros = jnp.zeros((VL,), jnp.float32)

    @pl.loop(0, NP // VL)
    def _init(i):
        off = i * VL
        h_v[pl.ds(off, VL)] = ones
        acc_v[pl.ds(off, VL)] = zeros

    @pl.loop(0, SWEEPS)
    def _sweep(_):
        @plsc.parallel_loop(0, EPT // VL, unroll=16)
        def _edges(e):
            off = e * VL
            s = src_v[pl.ds(off, VL)]
            d = dst_v[pl.ds(off, VL)]
            ww = w_v[pl.ds(off, VL)]
            hv = plsc.load_gather(h_v, [s])
            plsc.addupdate_scatter(acc_v, [d], hv * ww)

        pltpu.sync_copy(acc_v, sh_acc.at[wid])
        plsc.subcore_barrier()

        for t in range(NT):
            pltpu.sync_copy(sh_acc.at[t, pl.ds(nbase, NSL)], red_v.at[t])

        @pl.loop(0, NSL // VL)
        def _upd(j):
            off = j * VL
            a = red_v[0, pl.ds(off, VL)]
            for t in range(1, NT):
                a = a + red_v[t, pl.ds(off, VL)]
            hs_v[pl.ds(off, VL)] = a * (1.0 - ALPHA) + ALPHA

        pltpu.sync_copy(hs_v, sh_h.at[pl.ds(nbase, NSL)])

        @plsc.parallel_loop(0, NP // VL, unroll=8)
        def _clr(i):
            acc_v[pl.ds(i * VL, VL)] = zeros

        plsc.subcore_barrier()
        pltpu.sync_copy(sh_h, h_v)

    pltpu.sync_copy(hs_v, h_out.at[pl.ds(nbase, NSL)])


def _make_sweep():
    mesh = plsc.VectorSubcoreMesh(
        core_axis_name="c", subcore_axis_name="s", num_cores=1)
    return pl.kernel(
        _sweep_body,
        out_type=jax.ShapeDtypeStruct((NP,), jnp.float32),
        mesh=mesh,
        scratch_types=[
            pltpu.VMEM((EPT,), jnp.int32),
            pltpu.VMEM((EPT,), jnp.int32),
            pltpu.VMEM((EPT,), jnp.float32),
            pltpu.VMEM((NP,), jnp.float32),
            pltpu.VMEM((NP,), jnp.float32),
            pltpu.VMEM((NT, NSL), jnp.float32),
            pltpu.VMEM((NSL,), jnp.float32),
            pltpu.VMEM_SHARED((NT, NP), jnp.float32),
            pltpu.VMEM_SHARED((NP,), jnp.float32),
        ],
        compiler_params=pltpu.CompilerParams(needs_layout_passes=False),
    )


def _divide_tc(f_ref, h_ref, o_ref):
    o_ref[...] = f_ref[...] / h_ref[...]


def kernel(features, edge_index, edge_weight):
    h_pad = _make_sweep()(edge_index.reshape(2 * E), edge_weight)
    h = h_pad[:N].reshape(N, 1)

    return pl.pallas_call(
        _divide_tc,
        out_shape=jax.ShapeDtypeStruct((N, features.shape[1]), jnp.float32),
        grid=(10,),
        in_specs=[
            pl.BlockSpec((N // 10, 128), lambda i: (i, 0)),
            pl.BlockSpec((N // 10, 1), lambda i: (i, 0)),
        ],
        out_specs=pl.BlockSpec((N // 10, 128), lambda i: (i, 0)),
    )(features, h)


# speedup vs baseline: 127.4963x; 1.0161x over previous
import jax
import jax.numpy as jnp
from jax import lax
from jax.experimental import pallas as pl
from jax.experimental.pallas import tpu as pltpu
from jax.experimental.pallas import tpu_sc as plsc

N = 10000
E = 320000
ALPHA = 0.1
SWEEPS = 10
NT = 16
EPT = E // NT
VL = 16
NSL = 640
NP = NT * NSL


def _sweep_body(ei_hbm, w_hbm, h_out,
                src_v, dst_v, w_v, h_v, acc_v, red_v, hs_v, sh_acc, sh_h):
    wid = lax.axis_index("s")
    ebase = wid * EPT
    nbase = wid * NSL

    pltpu.sync_copy(ei_hbm.at[pl.ds(ebase, EPT)], src_v)
    pltpu.sync_copy(ei_hbm.at[pl.ds(E + ebase, EPT)], dst_v)
    pltpu.sync_copy(w_hbm.at[pl.ds(ebase, EPT)], w_v)

    ones = jnp.ones((VL,), jnp.float32)
    zeros = jnp.zeros((VL,), jnp.float32)

    @pl.loop(0, NP // VL)
    def _init(i):
        off = i * VL
        h_v[pl.ds(off, VL)] = ones
        acc_v[pl.ds(off, VL)] = zeros

    @pl.loop(0, EPT // VL)
    def _pack(i):
        off = i * VL
        s = src_v[pl.ds(off, VL)]
        d = dst_v[pl.ds(off, VL)]
        src_v[pl.ds(off, VL)] = s | (d << 14)

    @pl.loop(0, SWEEPS)
    def _sweep(_):
        @plsc.parallel_loop(0, EPT // VL, unroll=16)
        def _edges(e):
            off = e * VL
            p = src_v[pl.ds(off, VL)]
            ww = w_v[pl.ds(off, VL)]
            s = p & 0x3FFF
            d = lax.shift_right_logical(p, 14)
            hv = plsc.load_gather(h_v, [s])
            plsc.addupdate_scatter(acc_v, [d], hv * ww)

        pltpu.sync_copy(acc_v, sh_acc.at[wid])
        plsc.subcore_barrier()

        for t in range(NT):
            pltpu.sync_copy(sh_acc.at[t, pl.ds(nbase, NSL)], red_v.at[t])

        @pl.loop(0, NSL // VL)
        def _upd(j):
            off = j * VL
            a = red_v[0, pl.ds(off, VL)]
            for t in range(1, NT):
                a = a + red_v[t, pl.ds(off, VL)]
            hs_v[pl.ds(off, VL)] = a * (1.0 - ALPHA) + ALPHA

        pltpu.sync_copy(hs_v, sh_h.at[pl.ds(nbase, NSL)])

        @plsc.parallel_loop(0, NP // VL, unroll=8)
        def _clr(i):
            acc_v[pl.ds(i * VL, VL)] = zeros

        plsc.subcore_barrier()
        pltpu.sync_copy(sh_h, h_v)

    pltpu.sync_copy(hs_v, h_out.at[pl.ds(nbase, NSL)])


def _make_sweep():
    mesh = plsc.VectorSubcoreMesh(
        core_axis_name="c", subcore_axis_name="s", num_cores=1)
    return pl.kernel(
        _sweep_body,
        out_type=jax.ShapeDtypeStruct((NP,), jnp.float32),
        mesh=mesh,
        scratch_types=[
            pltpu.VMEM((EPT,), jnp.int32),
            pltpu.VMEM((EPT,), jnp.int32),
            pltpu.VMEM((EPT,), jnp.float32),
            pltpu.VMEM((NP,), jnp.float32),
            pltpu.VMEM((NP,), jnp.float32),
            pltpu.VMEM((NT, NSL), jnp.float32),
            pltpu.VMEM((NSL,), jnp.float32),
            pltpu.VMEM_SHARED((NT, NP), jnp.float32),
            pltpu.VMEM_SHARED((NP,), jnp.float32),
        ],
        compiler_params=pltpu.CompilerParams(needs_layout_passes=False),
    )


def _divide_tc(f_ref, h_ref, o_ref):
    o_ref[...] = f_ref[...] / h_ref[...]


def kernel(features, edge_index, edge_weight):
    h_pad = _make_sweep()(edge_index.reshape(2 * E), edge_weight)
    h = h_pad[:N].reshape(N, 1)

    return pl.pallas_call(
        _divide_tc,
        out_shape=jax.ShapeDtypeStruct((N, features.shape[1]), jnp.float32),
        grid=(10,),
        in_specs=[
            pl.BlockSpec((N // 10, 128), lambda i: (i, 0)),
            pl.BlockSpec((N // 10, 1), lambda i: (i, 0)),
        ],
        out_specs=pl.BlockSpec((N // 10, 128), lambda i: (i, 0)),
    )(features, h)


# speedup vs baseline: 141.2015x; 1.1075x over previous
import jax
import jax.numpy as jnp
from jax import lax
from jax.experimental import pallas as pl
from jax.experimental.pallas import tpu as pltpu
from jax.experimental.pallas import tpu_sc as plsc

N = 10000
E = 320000
ALPHA = 0.1
SWEEPS = 10
NT = 16
EPT = E // NT
VL = 16
NSL = 640
NP = NT * NSL


def _sweep_body(ei_hbm, w_hbm, h_out,
                src_v, dst_v, w_v, h_v, acc_v, red_v, hs_v, sh_acc, sh_h):
    wid = lax.axis_index("s")
    ebase = wid * EPT
    nbase = wid * NSL

    pltpu.sync_copy(ei_hbm.at[pl.ds(ebase, EPT)], src_v)
    pltpu.sync_copy(ei_hbm.at[pl.ds(E + ebase, EPT)], dst_v)
    pltpu.sync_copy(w_hbm.at[pl.ds(ebase, EPT)], w_v)

    ones = jnp.ones((VL,), jnp.float32)
    zeros = jnp.zeros((VL,), jnp.float32)

    @pl.loop(0, NP // VL)
    def _init(i):
        off = i * VL
        h_v[pl.ds(off, VL)] = ones
        acc_v[pl.ds(off, VL)] = zeros

    @pl.loop(0, SWEEPS)
    def _sweep(_):
        @plsc.parallel_loop(0, EPT // VL, unroll=16)
        def _edges(e):
            off = e * VL
            s = src_v[pl.ds(off, VL)]
            d = dst_v[pl.ds(off, VL)]
            ww = w_v[pl.ds(off, VL)]
            hv = plsc.load_gather(h_v, [s])
            plsc.addupdate_scatter(acc_v, [d], hv * ww)

        pltpu.sync_copy(acc_v, sh_acc.at[wid])
        plsc.subcore_barrier()

        pltpu.sync_copy(sh_acc.at[:, pl.ds(nbase, NSL)], red_v)

        @pl.loop(0, NSL // VL)
        def _upd(j):
            off = j * VL
            a = red_v[0, pl.ds(off, VL)]
            for t in range(1, NT):
                a = a + red_v[t, pl.ds(off, VL)]
            hs_v[pl.ds(off, VL)] = a * (1.0 - ALPHA) + ALPHA

        pltpu.sync_copy(hs_v, sh_h.at[pl.ds(nbase, NSL)])

        @plsc.parallel_loop(0, NP // VL, unroll=16)
        def _clr(i):
            acc_v[pl.ds(i * VL, VL)] = zeros

        plsc.subcore_barrier()
        pltpu.sync_copy(sh_h, h_v)

    pltpu.sync_copy(hs_v, h_out.at[pl.ds(nbase, NSL)])


def _make_sweep():
    mesh = plsc.VectorSubcoreMesh(
        core_axis_name="c", subcore_axis_name="s", num_cores=1)
    return pl.kernel(
        _sweep_body,
        out_type=jax.ShapeDtypeStruct((NP,), jnp.float32),
        mesh=mesh,
        scratch_types=[
            pltpu.VMEM((EPT,), jnp.int32),
            pltpu.VMEM((EPT,), jnp.int32),
            pltpu.VMEM((EPT,), jnp.float32),
            pltpu.VMEM((NP,), jnp.float32),
            pltpu.VMEM((NP,), jnp.float32),
            pltpu.VMEM((NT, NSL), jnp.float32),
            pltpu.VMEM((NSL,), jnp.float32),
            pltpu.VMEM_SHARED((NT, NP), jnp.float32),
            pltpu.VMEM_SHARED((NP,), jnp.float32),
        ],
        compiler_params=pltpu.CompilerParams(needs_layout_passes=False),
    )


def _divide_tc(f_ref, h_ref, o_ref):
    o_ref[...] = f_ref[...] / h_ref[...]


def kernel(features, edge_index, edge_weight):
    h_pad = _make_sweep()(edge_index.reshape(2 * E), edge_weight)
    h = h_pad[:N].reshape(N, 1)

    return pl.pallas_call(
        _divide_tc,
        out_shape=jax.ShapeDtypeStruct((N, features.shape[1]), jnp.float32),
        grid=(10,),
        in_specs=[
            pl.BlockSpec((N // 10, 128), lambda i: (i, 0)),
            pl.BlockSpec((N // 10, 1), lambda i: (i, 0)),
        ],
        out_specs=pl.BlockSpec((N // 10, 128), lambda i: (i, 0)),
    )(features, h)


# speedup vs baseline: 150.3722x; 1.0649x over previous
import jax
import jax.numpy as jnp
from jax import lax
from jax.experimental import pallas as pl
from jax.experimental.pallas import tpu as pltpu
from jax.experimental.pallas import tpu_sc as plsc

N = 10000
E = 320000
ALPHA = 0.1
SWEEPS = 10
NT = 16
EPT = E // NT
VL = 16
NSL = 640
NP = NT * NSL


def _sweep_body(ei_hbm, w_hbm, h_out,
                src_v, dst_v, w_v, h_v, acc_v, red_v, hs_v, sh_acc, sh_h,
                sem):
    wid = lax.axis_index("s")
    ebase = wid * EPT
    nbase = wid * NSL

    cp1 = pltpu.async_copy(ei_hbm.at[pl.ds(ebase, EPT)], src_v, sem)
    cp2 = pltpu.async_copy(ei_hbm.at[pl.ds(E + ebase, EPT)], dst_v, sem)
    cp3 = pltpu.async_copy(w_hbm.at[pl.ds(ebase, EPT)], w_v, sem)

    ones = jnp.ones((VL,), jnp.float32)
    zeros = jnp.zeros((VL,), jnp.float32)

    @pl.loop(0, NP // VL)
    def _init(i):
        off = i * VL
        h_v[pl.ds(off, VL)] = ones
        acc_v[pl.ds(off, VL)] = zeros

    cp1.wait()
    cp2.wait()
    cp3.wait()

    @pl.loop(0, SWEEPS)
    def _sweep(_):
        @plsc.parallel_loop(0, EPT // VL, unroll=16)
        def _edges(e):
            off = e * VL
            s = src_v[pl.ds(off, VL)]
            d = dst_v[pl.ds(off, VL)]
            ww = w_v[pl.ds(off, VL)]
            hv = plsc.load_gather(h_v, [s])
            plsc.addupdate_scatter(acc_v, [d], hv * ww)

        pltpu.sync_copy(acc_v, sh_acc.at[wid])
        plsc.subcore_barrier()

        rd = pltpu.async_copy(sh_acc.at[:, pl.ds(nbase, NSL)], red_v, sem)

        @plsc.parallel_loop(0, NP // VL, unroll=16)
        def _clr(i):
            acc_v[pl.ds(i * VL, VL)] = zeros

        rd.wait()

        @pl.loop(0, NSL // VL)
        def _upd(j):
            off = j * VL
            a = red_v[0, pl.ds(off, VL)]
            for t in range(1, NT):
                a = a + red_v[t, pl.ds(off, VL)]
            hs_v[pl.ds(off, VL)] = a * (1.0 - ALPHA) + ALPHA

        pltpu.sync_copy(hs_v, sh_h.at[pl.ds(nbase, NSL)])
        plsc.subcore_barrier()
        pltpu.sync_copy(sh_h, h_v)

    pltpu.sync_copy(hs_v, h_out.at[pl.ds(nbase, NSL)])


def _make_sweep():
    mesh = plsc.VectorSubcoreMesh(
        core_axis_name="c", subcore_axis_name="s", num_cores=1)
    return pl.kernel(
        _sweep_body,
        out_type=jax.ShapeDtypeStruct((NP,), jnp.float32),
        mesh=mesh,
        scratch_types=[
            pltpu.VMEM((EPT,), jnp.int32),
            pltpu.VMEM((EPT,), jnp.int32),
            pltpu.VMEM((EPT,), jnp.float32),
            pltpu.VMEM((NP,), jnp.float32),
            pltpu.VMEM((NP,), jnp.float32),
            pltpu.VMEM((NT, NSL), jnp.float32),
            pltpu.VMEM((NSL,), jnp.float32),
            pltpu.VMEM_SHARED((NT, NP), jnp.float32),
            pltpu.VMEM_SHARED((NP,), jnp.float32),
            pltpu.SemaphoreType.DMA,
        ],
        compiler_params=pltpu.CompilerParams(needs_layout_passes=False),
    )


def _divide_tc(f_ref, h_ref, o_ref):
    o_ref[...] = f_ref[...] / h_ref[...]


def kernel(features, edge_index, edge_weight):
    h_pad = _make_sweep()(edge_index.reshape(2 * E), edge_weight)
    h = h_pad[:N].reshape(N, 1)

    return pl.pallas_call(
        _divide_tc,
        out_shape=jax.ShapeDtypeStruct((N, features.shape[1]), jnp.float32),
        grid=(10,),
        in_specs=[
            pl.BlockSpec((N // 10, 128), lambda i: (i, 0)),
            pl.BlockSpec((N // 10, 1), lambda i: (i, 0)),
        ],
        out_specs=pl.BlockSpec((N // 10, 128), lambda i: (i, 0)),
    )(features, h)
